# Initial kernel scaffold; baseline (speedup 1.0000x reference)
import jax
import jax.numpy as jnp
from jax.experimental import pallas as pl


def kernel(features, visible, img_label, memory, clutter_bank, lru):
    raise NotImplementedError("write your pallas kernel here")


# speedup vs baseline: 3.4620x; 3.4620x over previous
import jax
import jax.numpy as jnp
from jax.experimental import pallas as pl
from jax.experimental.pallas import tpu as pltpu

_MAX_N = 512
_N_CLASSES = 64
_MESH_DIM = 128
_NUM_CLUTTER = 5
_BANK_SIZE = 4096
_MOMENTUM = 0.9
_TILE = 512
_MEM_TILES = _N_CLASSES
_CLUT_TILES = _NUM_CLUTTER * _BANK_SIZE // _TILE
_GRID = _MEM_TILES + _CLUT_TILES


def _body(w_smem, off_smem, x1_ref, x2_ref, mem_ref, clut_ref, vismt_ref,
          noiset_ref, out1_ref, out2_ref, out3_ref, out4_ref):
    t = pl.program_id(0)
    tmem = jnp.minimum(t, _MEM_TILES - 1)
    tclut = jnp.maximum(t - _MEM_TILES, 0)

    w_mem = mem_ref[0]
    x1 = x1_ref[...]
    x2 = x2_ref[...]

    @pl.when(t < _MEM_TILES)
    def _():
        out1_ref[...] = jnp.dot(x1, w_mem, preferred_element_type=jnp.float32)

    @pl.when(t >= _MEM_TILES)
    def _():
        out1_ref[...] = jnp.dot(x1, clut_ref[...],
                                preferred_element_type=jnp.float32)

    out2_ref[...] = jnp.dot(x2, w_mem, preferred_element_type=jnp.float32)

    base = tmem * 8
    vk = (w_smem[base + 0] * vismt_ref[0] + w_smem[base + 1] * vismt_ref[1]
          + w_smem[base + 2] * vismt_ref[2] + w_smem[base + 3] * vismt_ref[3])
    blend = w_smem[base + 4] * w_mem + vk
    n2 = jnp.sum(blend * blend, axis=0, keepdims=True)
    inv = 1.0 / jnp.maximum(jnp.sqrt(n2), 1e-12)
    out4_ref[0] = blend * inv

    off = off_smem[0] - tclut * _TILE
    li2 = jax.lax.broadcasted_iota(jnp.int32, (32, _TILE), 1)
    ri2 = jax.lax.broadcasted_iota(jnp.int32, (32, _TILE), 0)
    sel = (li2 - off == ri2).astype(jnp.float32)
    scat = jnp.dot(noiset_ref[...], sel, preferred_element_type=jnp.float32)
    ci = jax.lax.broadcasted_iota(jnp.int32, (1, _TILE), 1)
    cm = (ci >= off) & (ci < off + _NUM_CLUTTER * 4)
    upd = jnp.where(cm, scat, clut_ref[...])
    s2 = jnp.sum(upd * upd, axis=0, keepdims=True)
    cinv = 1.0 / jnp.maximum(jnp.sqrt(s2), 1e-12)
    out3_ref[...] = upd * cinv


def kernel(features, visible, img_label, memory, clutter_bank, lru):
    B = features.shape[0]
    C, V, J = memory.shape
    NC = _NUM_CLUTTER
    f32 = jnp.float32

    vertices = features[:, :_MAX_N, :]
    noise = features[:, _MAX_N:, :]
    x1 = vertices.reshape(B * _MAX_N, V)
    x2 = jnp.zeros((32, V), f32).at[: B * NC].set(noise.reshape(B * NC, V))
    noiset = x2.T

    oh = (img_label[:, None] == jnp.arange(C)[None, :]).astype(f32)
    count = jnp.sum(oh, axis=0)
    wmat = oh.T * jnp.where(count > 0, (1.0 - _MOMENTUM)
                            / jnp.maximum(count, 1.0), 0.0)[:, None]
    alpha = jnp.where(count > 0, _MOMENTUM, 1.0)
    wflat = jnp.concatenate(
        [wmat, alpha[:, None], jnp.zeros((C, 3), f32)], axis=1).reshape(-1)

    vismt = (vertices * visible[..., None].astype(f32)).transpose(0, 2, 1)

    new_lru = (lru + 1) % (_BANK_SIZE // B)
    off = jnp.asarray(new_lru * NC * B, jnp.int32).reshape(1)

    ncols = C * J + NC * _BANK_SIZE
    out_shapes = (
        jax.ShapeDtypeStruct((B * _MAX_N, ncols), f32),
        jax.ShapeDtypeStruct((32, C * J), f32),
        jax.ShapeDtypeStruct((V, NC * _BANK_SIZE), f32),
        jax.ShapeDtypeStruct((C, V, J), f32),
    )
    out1, out2, out3, out4 = pl.pallas_call(
        _body,
        out_shape=out_shapes,
        grid=(_GRID,),
        in_specs=[
            pl.BlockSpec(memory_space=pltpu.SMEM),
            pl.BlockSpec(memory_space=pltpu.SMEM),
            pl.BlockSpec((B * _MAX_N, V), lambda t: (0, 0)),
            pl.BlockSpec((32, V), lambda t: (0, 0)),
            pl.BlockSpec((1, V, J), lambda t: (jnp.minimum(t, _MEM_TILES - 1), 0, 0)),
            pl.BlockSpec((V, _TILE), lambda t: (0, jnp.maximum(t - _MEM_TILES, 0))),
            pl.BlockSpec((B, V, J), lambda t: (0, 0, 0)),
            pl.BlockSpec((V, 32), lambda t: (0, 0)),
        ],
        out_specs=[
            pl.BlockSpec((B * _MAX_N, _TILE), lambda t: (0, t)),
            pl.BlockSpec((32, _TILE), lambda t: (0, jnp.minimum(t, _MEM_TILES - 1))),
            pl.BlockSpec((V, _TILE), lambda t: (0, jnp.maximum(t - _MEM_TILES, 0))),
            pl.BlockSpec((1, V, J), lambda t: (jnp.minimum(t, _MEM_TILES - 1), 0, 0)),
        ],
        compiler_params=pltpu.CompilerParams(
            dimension_semantics=("parallel",),
        ),
        name="mesh_memory_fused",
    )(wflat, off, x1, x2, memory, clutter_bank, vismt, noiset)

    similarity = out1.reshape(B, _MAX_N, ncols)
    noise_similarity = out2[: B * NC].reshape(B, NC, C * J)
    return similarity, noise_similarity, out4, out3


# speedup vs baseline: 3.5674x; 1.0304x over previous
import jax
import jax.numpy as jnp
from jax.experimental import pallas as pl
from jax.experimental.pallas import tpu as pltpu

_MAX_N = 512
_N_CLASSES = 64
_MESH_DIM = 128
_NUM_CLUTTER = 5
_BANK_SIZE = 4096
_MOMENTUM = 0.9
_TILE = 512
_MEM_TILES = _N_CLASSES
_CLUT_TILES = _NUM_CLUTTER * _BANK_SIZE // _TILE
_GRID = _MEM_TILES + _CLUT_TILES


def _body(w_smem, off_smem, x1_ref, x2_ref, mem_ref, clut_ref, vismt_ref,
          noiset_ref, out1_ref, out2_ref, out3_ref, out4_ref):
    t = pl.program_id(0)

    @pl.when(t < _MEM_TILES)
    def _():
        w_mem = mem_ref[0]
        out1_ref[...] = jnp.dot(x1_ref[...], w_mem,
                                preferred_element_type=jnp.float32)
        out2_ref[...] = jnp.dot(x2_ref[...], w_mem,
                                preferred_element_type=jnp.float32)
        base = t * 8
        vk = (w_smem[base + 0] * vismt_ref[0]
              + w_smem[base + 1] * vismt_ref[1]
              + w_smem[base + 2] * vismt_ref[2]
              + w_smem[base + 3] * vismt_ref[3])
        blend = w_smem[base + 4] * w_mem + vk
        n2 = jnp.sum(blend * blend, axis=0, keepdims=True)
        inv = 1.0 / jnp.maximum(jnp.sqrt(n2), 1e-12)
        out4_ref[0] = blend * inv

    @pl.when(t >= _MEM_TILES)
    def _():
        cl = clut_ref[...]
        out1_ref[...] = jnp.dot(x1_ref[...], cl,
                                preferred_element_type=jnp.float32)
        off = off_smem[0] - (t - _MEM_TILES) * _TILE
        li2 = jax.lax.broadcasted_iota(jnp.int32, (32, _TILE), 1)
        ri2 = jax.lax.broadcasted_iota(jnp.int32, (32, _TILE), 0)
        sel = (li2 - off == ri2).astype(jnp.float32)
        scat = jnp.dot(noiset_ref[...], sel,
                       preferred_element_type=jnp.float32)
        ci = jax.lax.broadcasted_iota(jnp.int32, (1, _TILE), 1)
        cm = (ci >= off) & (ci < off + _NUM_CLUTTER * 4)
        upd = jnp.where(cm, scat, cl)
        s2 = jnp.sum(upd * upd, axis=0, keepdims=True)
        cinv = 1.0 / jnp.maximum(jnp.sqrt(s2), 1e-12)
        out3_ref[...] = upd * cinv


def kernel(features, visible, img_label, memory, clutter_bank, lru):
    B = features.shape[0]
    C, V, J = memory.shape
    NC = _NUM_CLUTTER
    f32 = jnp.float32

    vertices = features[:, :_MAX_N, :]
    noise = features[:, _MAX_N:, :]
    x1 = vertices.reshape(B * _MAX_N, V)
    x2 = jnp.zeros((32, V), f32).at[: B * NC].set(noise.reshape(B * NC, V))
    noiset = x2.T

    oh = (img_label[:, None] == jnp.arange(C)[None, :]).astype(f32)
    count = jnp.sum(oh, axis=0)
    wmat = oh.T * jnp.where(count > 0, (1.0 - _MOMENTUM)
                            / jnp.maximum(count, 1.0), 0.0)[:, None]
    alpha = jnp.where(count > 0, _MOMENTUM, 1.0)
    wflat = jnp.concatenate(
        [wmat, alpha[:, None], jnp.zeros((C, 3), f32)], axis=1).reshape(-1)

    vismt = (vertices * visible[..., None].astype(f32)).transpose(0, 2, 1)

    new_lru = (lru + 1) % (_BANK_SIZE // B)
    off = jnp.asarray(new_lru * NC * B, jnp.int32).reshape(1)

    ncols = C * J + NC * _BANK_SIZE
    out_shapes = (
        jax.ShapeDtypeStruct((B * _MAX_N, ncols), f32),
        jax.ShapeDtypeStruct((32, C * J), f32),
        jax.ShapeDtypeStruct((V, NC * _BANK_SIZE), f32),
        jax.ShapeDtypeStruct((C, V, J), f32),
    )
    out1, out2, out3, out4 = pl.pallas_call(
        _body,
        out_shape=out_shapes,
        grid=(_GRID,),
        in_specs=[
            pl.BlockSpec(memory_space=pltpu.SMEM),
            pl.BlockSpec(memory_space=pltpu.SMEM),
            pl.BlockSpec((B * _MAX_N, V), lambda t: (0, 0)),
            pl.BlockSpec((32, V), lambda t: (0, 0)),
            pl.BlockSpec((1, V, J), lambda t: (jnp.minimum(t, _MEM_TILES - 1), 0, 0)),
            pl.BlockSpec((V, _TILE), lambda t: (0, jnp.maximum(t - _MEM_TILES, 0))),
            pl.BlockSpec((B, V, J), lambda t: (0, 0, 0)),
            pl.BlockSpec((V, 32), lambda t: (0, 0)),
        ],
        out_specs=[
            pl.BlockSpec((B * _MAX_N, _TILE), lambda t: (0, t)),
            pl.BlockSpec((32, _TILE), lambda t: (0, jnp.minimum(t, _MEM_TILES - 1))),
            pl.BlockSpec((V, _TILE), lambda t: (0, jnp.maximum(t - _MEM_TILES, 0))),
            pl.BlockSpec((1, V, J), lambda t: (jnp.minimum(t, _MEM_TILES - 1), 0, 0)),
        ],
        compiler_params=pltpu.CompilerParams(
            dimension_semantics=("arbitrary",),
        ),
        name="mesh_memory_fused",
    )(wflat, off, x1, x2, memory, clutter_bank, vismt, noiset)

    similarity = out1.reshape(B, _MAX_N, ncols)
    noise_similarity = out2[: B * NC].reshape(B, NC, C * J)
    return similarity, noise_similarity, out4, out3


# speedup vs baseline: 4.0531x; 1.1362x over previous
import jax
import jax.numpy as jnp
from jax.experimental import pallas as pl
from jax.experimental.pallas import tpu as pltpu

_MAX_N = 512
_N_CLASSES = 64
_MESH_DIM = 128
_NUM_CLUTTER = 5
_BANK_SIZE = 4096
_MOMENTUM = 0.9
_TILE = 1024
_CPT = _TILE // 512
_MEM_TILES = _N_CLASSES // _CPT
_CLUT_TILES = _NUM_CLUTTER * _BANK_SIZE // _TILE
_GRID = _MEM_TILES + _CLUT_TILES


def _body(w_smem, off_smem, x1_ref, x2_ref, mem_ref, clut_ref, vismt_ref,
          noiset_ref, out1_ref, out2_ref, out3_ref, out4_ref):
    t = pl.program_id(0)

    @pl.when(t < _MEM_TILES)
    def _():
        x1 = x1_ref[...]
        x2 = x2_ref[...]
        for k in range(_CPT):
            w_mem = mem_ref[k]
            cs = slice(k * 512, (k + 1) * 512)
            out1_ref[:, cs] = jnp.dot(x1, w_mem,
                                      preferred_element_type=jnp.float32)
            out2_ref[:, cs] = jnp.dot(x2, w_mem,
                                      preferred_element_type=jnp.float32)
            base = (t * _CPT + k) * 8
            vk = (w_smem[base + 0] * vismt_ref[0]
                  + w_smem[base + 1] * vismt_ref[1]
                  + w_smem[base + 2] * vismt_ref[2]
                  + w_smem[base + 3] * vismt_ref[3])
            blend = w_smem[base + 4] * w_mem + vk
            n2 = jnp.sum(blend * blend, axis=0, keepdims=True)
            inv = 1.0 / jnp.maximum(jnp.sqrt(n2), 1e-12)
            out4_ref[k] = blend * inv

    @pl.when(t >= _MEM_TILES)
    def _():
        cl = clut_ref[...]
        out1_ref[...] = jnp.dot(x1_ref[...], cl,
                                preferred_element_type=jnp.float32)
        off = off_smem[0] - (t - _MEM_TILES) * _TILE
        li2 = jax.lax.broadcasted_iota(jnp.int32, (32, _TILE), 1)
        ri2 = jax.lax.broadcasted_iota(jnp.int32, (32, _TILE), 0)
        sel = (li2 - off == ri2).astype(jnp.float32)
        scat = jnp.dot(noiset_ref[...], sel,
                       preferred_element_type=jnp.float32)
        ci = jax.lax.broadcasted_iota(jnp.int32, (1, _TILE), 1)
        cm = (ci >= off) & (ci < off + _NUM_CLUTTER * 4)
        upd = jnp.where(cm, scat, cl)
        s2 = jnp.sum(upd * upd, axis=0, keepdims=True)
        cinv = 1.0 / jnp.maximum(jnp.sqrt(s2), 1e-12)
        out3_ref[...] = upd * cinv


def kernel(features, visible, img_label, memory, clutter_bank, lru):
    B = features.shape[0]
    C, V, J = memory.shape
    NC = _NUM_CLUTTER
    f32 = jnp.float32

    vertices = features[:, :_MAX_N, :]
    noise = features[:, _MAX_N:, :]
    x1 = vertices.reshape(B * _MAX_N, V)
    x2 = jnp.zeros((32, V), f32).at[: B * NC].set(noise.reshape(B * NC, V))
    noiset = x2.T

    oh = (img_label[:, None] == jnp.arange(C)[None, :]).astype(f32)
    count = jnp.sum(oh, axis=0)
    wmat = oh.T * jnp.where(count > 0, (1.0 - _MOMENTUM)
                            / jnp.maximum(count, 1.0), 0.0)[:, None]
    alpha = jnp.where(count > 0, _MOMENTUM, 1.0)
    wflat = jnp.concatenate(
        [wmat, alpha[:, None], jnp.zeros((C, 3), f32)], axis=1).reshape(-1)

    vismt = (vertices * visible[..., None].astype(f32)).transpose(0, 2, 1)

    new_lru = (lru + 1) % (_BANK_SIZE // B)
    off = jnp.asarray(new_lru * NC * B, jnp.int32).reshape(1)

    ncols = C * J + NC * _BANK_SIZE
    out_shapes = (
        jax.ShapeDtypeStruct((B * _MAX_N, ncols), f32),
        jax.ShapeDtypeStruct((32, C * J), f32),
        jax.ShapeDtypeStruct((V, NC * _BANK_SIZE), f32),
        jax.ShapeDtypeStruct((C, V, J), f32),
    )
    out1, out2, out3, out4 = pl.pallas_call(
        _body,
        out_shape=out_shapes,
        grid=(_GRID,),
        in_specs=[
            pl.BlockSpec(memory_space=pltpu.SMEM),
            pl.BlockSpec(memory_space=pltpu.SMEM),
            pl.BlockSpec((B * _MAX_N, V), lambda t: (0, 0)),
            pl.BlockSpec((32, V), lambda t: (0, 0)),
            pl.BlockSpec((_CPT, V, J), lambda t: (jnp.minimum(t, _MEM_TILES - 1), 0, 0)),
            pl.BlockSpec((V, _TILE), lambda t: (0, jnp.maximum(t - _MEM_TILES, 0))),
            pl.BlockSpec((B, V, J), lambda t: (0, 0, 0)),
            pl.BlockSpec((V, 32), lambda t: (0, 0)),
        ],
        out_specs=[
            pl.BlockSpec((B * _MAX_N, _TILE), lambda t: (0, t)),
            pl.BlockSpec((32, _TILE), lambda t: (0, jnp.minimum(t, _MEM_TILES - 1))),
            pl.BlockSpec((V, _TILE), lambda t: (0, jnp.maximum(t - _MEM_TILES, 0))),
            pl.BlockSpec((_CPT, V, J), lambda t: (jnp.minimum(t, _MEM_TILES - 1), 0, 0)),
        ],
        compiler_params=pltpu.CompilerParams(
            dimension_semantics=("arbitrary",),
        ),
        name="mesh_memory_fused",
    )(wflat, off, x1, x2, memory, clutter_bank, vismt, noiset)

    similarity = out1.reshape(B, _MAX_N, ncols)
    noise_similarity = out2[: B * NC].reshape(B, NC, C * J)
    return similarity, noise_similarity, out4, out3


# speedup vs baseline: 4.2084x; 1.0383x over previous
import jax
import jax.numpy as jnp
from jax.experimental import pallas as pl
from jax.experimental.pallas import tpu as pltpu

_MAX_N = 512
_N_CLASSES = 64
_MESH_DIM = 128
_NUM_CLUTTER = 5
_BANK_SIZE = 4096
_MOMENTUM = 0.9
_TILE = 2048
_CPT = _TILE // 512
_MEM_TILES = _N_CLASSES // _CPT
_CLUT_TILES = _NUM_CLUTTER * _BANK_SIZE // _TILE
_GRID = _MEM_TILES + _CLUT_TILES


def _body(w_smem, off_smem, x1_ref, x2_ref, mem_ref, clut_ref, vismt_ref,
          noiset_ref, out1_ref, out2_ref, out3_ref, out4_ref):
    t = pl.program_id(0)

    @pl.when(t < _MEM_TILES)
    def _():
        x1 = x1_ref[...]
        x2 = x2_ref[...]
        for k in range(_CPT):
            w_mem = mem_ref[k]
            cs = slice(k * 512, (k + 1) * 512)
            out1_ref[:, cs] = jnp.dot(x1, w_mem,
                                      preferred_element_type=jnp.float32)
            out2_ref[:, cs] = jnp.dot(x2, w_mem,
                                      preferred_element_type=jnp.float32)
            base = (t * _CPT + k) * 8
            vk = (w_smem[base + 0] * vismt_ref[0]
                  + w_smem[base + 1] * vismt_ref[1]
                  + w_smem[base + 2] * vismt_ref[2]
                  + w_smem[base + 3] * vismt_ref[3])
            blend = w_smem[base + 4] * w_mem + vk
            n2 = jnp.sum(blend * blend, axis=0, keepdims=True)
            inv = 1.0 / jnp.maximum(jnp.sqrt(n2), 1e-12)
            out4_ref[k] = blend * inv

    @pl.when(t >= _MEM_TILES)
    def _():
        cl = clut_ref[...]
        out1_ref[...] = jnp.dot(x1_ref[...], cl,
                                preferred_element_type=jnp.float32)
        off = off_smem[0] - (t - _MEM_TILES) * _TILE
        li2 = jax.lax.broadcasted_iota(jnp.int32, (32, _TILE), 1)
        ri2 = jax.lax.broadcasted_iota(jnp.int32, (32, _TILE), 0)
        sel = (li2 - off == ri2).astype(jnp.float32)
        scat = jnp.dot(noiset_ref[...], sel,
                       preferred_element_type=jnp.float32)
        ci = jax.lax.broadcasted_iota(jnp.int32, (1, _TILE), 1)
        cm = (ci >= off) & (ci < off + _NUM_CLUTTER * 4)
        upd = jnp.where(cm, scat, cl)
        s2 = jnp.sum(upd * upd, axis=0, keepdims=True)
        cinv = 1.0 / jnp.maximum(jnp.sqrt(s2), 1e-12)
        out3_ref[...] = upd * cinv


def kernel(features, visible, img_label, memory, clutter_bank, lru):
    B = features.shape[0]
    C, V, J = memory.shape
    NC = _NUM_CLUTTER
    f32 = jnp.float32

    vertices = features[:, :_MAX_N, :]
    noise = features[:, _MAX_N:, :]
    x1 = vertices.reshape(B * _MAX_N, V)
    x2 = jnp.zeros((32, V), f32).at[: B * NC].set(noise.reshape(B * NC, V))
    noiset = x2.T

    oh = (img_label[:, None] == jnp.arange(C)[None, :]).astype(f32)
    count = jnp.sum(oh, axis=0)
    wmat = oh.T * jnp.where(count > 0, (1.0 - _MOMENTUM)
                            / jnp.maximum(count, 1.0), 0.0)[:, None]
    alpha = jnp.where(count > 0, _MOMENTUM, 1.0)
    wflat = jnp.concatenate(
        [wmat, alpha[:, None], jnp.zeros((C, 3), f32)], axis=1).reshape(-1)

    vismt = (vertices * visible[..., None].astype(f32)).transpose(0, 2, 1)

    new_lru = (lru + 1) % (_BANK_SIZE // B)
    off = jnp.asarray(new_lru * NC * B, jnp.int32).reshape(1)

    ncols = C * J + NC * _BANK_SIZE
    out_shapes = (
        jax.ShapeDtypeStruct((B * _MAX_N, ncols), f32),
        jax.ShapeDtypeStruct((32, C * J), f32),
        jax.ShapeDtypeStruct((V, NC * _BANK_SIZE), f32),
        jax.ShapeDtypeStruct((C, V, J), f32),
    )
    out1, out2, out3, out4 = pl.pallas_call(
        _body,
        out_shape=out_shapes,
        grid=(_GRID,),
        in_specs=[
            pl.BlockSpec(memory_space=pltpu.SMEM),
            pl.BlockSpec(memory_space=pltpu.SMEM),
            pl.BlockSpec((B * _MAX_N, V), lambda t: (0, 0)),
            pl.BlockSpec((32, V), lambda t: (0, 0)),
            pl.BlockSpec((_CPT, V, J), lambda t: (jnp.minimum(t, _MEM_TILES - 1), 0, 0)),
            pl.BlockSpec((V, _TILE), lambda t: (0, jnp.maximum(t - _MEM_TILES, 0))),
            pl.BlockSpec((B, V, J), lambda t: (0, 0, 0)),
            pl.BlockSpec((V, 32), lambda t: (0, 0)),
        ],
        out_specs=[
            pl.BlockSpec((B * _MAX_N, _TILE), lambda t: (0, t)),
            pl.BlockSpec((32, _TILE), lambda t: (0, jnp.minimum(t, _MEM_TILES - 1))),
            pl.BlockSpec((V, _TILE), lambda t: (0, jnp.maximum(t - _MEM_TILES, 0))),
            pl.BlockSpec((_CPT, V, J), lambda t: (jnp.minimum(t, _MEM_TILES - 1), 0, 0)),
        ],
        compiler_params=pltpu.CompilerParams(
            dimension_semantics=("arbitrary",),
            vmem_limit_bytes=52 * 1024 * 1024,
        ),
        name="mesh_memory_fused",
    )(wflat, off, x1, x2, memory, clutter_bank, vismt, noiset)

    similarity = out1.reshape(B, _MAX_N, ncols)
    noise_similarity = out2[: B * NC].reshape(B, NC, C * J)
    return similarity, noise_similarity, out4, out3
